# baseline (speedup 1.0000x reference)
import numpy as np
import jax
import jax.numpy as jnp
from jax.experimental import pallas as pl

F32 = jnp.float32
FRAMES = 8
SEGMENTS = 16
EPS = 1e-5


def _mel_body(x_ref, w_ref, b_ref, caw_ref, cab_ref, feat_ref, pred_ref):
    x = x_ref[0]
    y4 = jnp.dot(x, w_ref[...], preferred_element_type=F32) + b_ref[0]
    feat_ref[0] = y4
    pooled = y4.mean(axis=0).reshape(1, 1024)
    pred_ref[0, 0] = (
        jnp.dot(pooled, caw_ref[...], preferred_element_type=F32)[0]
        + cab_ref[0]
    )


def _mel_call(audio, mel_w, mel_b, cls_a_w, cls_a_b):
    return pl.pallas_call(
        _mel_body,
        grid=(SEGMENTS,),
        in_specs=[
            pl.BlockSpec((1, 125, 64), lambda i: (i, 0, 0)),
            pl.BlockSpec((64, 1024), lambda i: (0, 0)),
            pl.BlockSpec((1, 1024), lambda i: (0, 0)),
            pl.BlockSpec((1024, 15), lambda i: (0, 0)),
            pl.BlockSpec((1, 15), lambda i: (0, 0)),
        ],
        out_specs=[
            pl.BlockSpec((1, 125, 1024), lambda i: (i, 0, 0)),
            pl.BlockSpec((1, 1, 15), lambda i: (i, 0, 0)),
        ],
        out_shape=[
            jax.ShapeDtypeStruct((SEGMENTS, 125, 1024), F32),
            jax.ShapeDtypeStruct((SEGMENTS, 1, 15), F32),
        ],
    )(audio, mel_w, mel_b, cls_a_w, cls_a_b)


def _patch_body(x_ref, w_ref, b_ref, cw_ref, cb_ref, fv_ref, cam_ref, pv_ref):
    x = x_ref[...]
    v = jnp.dot(x, w_ref[...], preferred_element_type=F32) + b_ref[0]
    v = jnp.maximum(v, 0.0)
    fv_ref[...] = v
    cam = jnp.dot(v, cw_ref[...], preferred_element_type=F32) + cb_ref[0]
    pv_ref[0] = cam.reshape(8, 49, 15).mean(axis=1)
    cam_ref[...] = jnp.maximum(cam, 0.0)


def _patch_call(patches, patch_w, patch_b, cls_v_w, cls_v_b):
    return pl.pallas_call(
        _patch_body,
        grid=(16,),
        in_specs=[
            pl.BlockSpec((392, 3072), lambda i: (i, 0)),
            pl.BlockSpec((3072, 512), lambda i: (0, 0)),
            pl.BlockSpec((1, 512), lambda i: (0, 0)),
            pl.BlockSpec((512, 15), lambda i: (0, 0)),
            pl.BlockSpec((1, 15), lambda i: (0, 0)),
        ],
        out_specs=[
            pl.BlockSpec((392, 512), lambda i: (i, 0)),
            pl.BlockSpec((392, 15), lambda i: (i, 0)),
            pl.BlockSpec((1, 8, 15), lambda i: (i, 0, 0)),
        ],
        out_shape=[
            jax.ShapeDtypeStruct((6272, 512), F32),
            jax.ShapeDtypeStruct((6272, 15), F32),
            jax.ShapeDtypeStruct((16, 8, 15), F32),
        ],
    )(patches, patch_w, patch_b, cls_v_w, cls_v_b)


def _spaconv_body(x_ref, w_ref, y_ref, sum_ref, sq_ref):
    xp = x_ref[...]
    acc = jnp.zeros((392, 512), F32)
    for k in range(9):
        dh, dw = k // 3, k % 3
        xs = xp[:, dh:dh + 7, dw:dw + 7, :].reshape(392, 512)
        acc = acc + jnp.dot(xs, w_ref[k], preferred_element_type=F32)
    y_ref[...] = acc.reshape(8, 7, 7, 512)
    sum_ref[0, 0] = acc.sum(axis=0)
    sq_ref[0, 0] = (acc * acc).sum(axis=0)


def _spaconv_call(x_pad, w9):
    return pl.pallas_call(
        _spaconv_body,
        grid=(16,),
        in_specs=[
            pl.BlockSpec((8, 9, 9, 512), lambda i: (i, 0, 0, 0)),
            pl.BlockSpec((9, 512, 512), lambda i: (0, 0, 0)),
        ],
        out_specs=[
            pl.BlockSpec((8, 7, 7, 512), lambda i: (i, 0, 0, 0)),
            pl.BlockSpec((1, 1, 512), lambda i: (i, 0, 0)),
            pl.BlockSpec((1, 1, 512), lambda i: (i, 0, 0)),
        ],
        out_shape=[
            jax.ShapeDtypeStruct((128, 7, 7, 512), F32),
            jax.ShapeDtypeStruct((16, 1, 512), F32),
            jax.ShapeDtypeStruct((16, 1, 512), F32),
        ],
    )(x_pad, w9)


def _spabn_body(y_ref, sc_ref, sh_ref, out_ref, emb_ref):
    y = y_ref[...]
    h = jnp.maximum(y * sc_ref[0] + sh_ref[0], 0.0)
    out_ref[...] = h
    emb_ref[...] = h.reshape(8, 49, 512).max(axis=1)


def _spabn_call(y, scale, shift):
    return pl.pallas_call(
        _spabn_body,
        grid=(16,),
        in_specs=[
            pl.BlockSpec((8, 7, 7, 512), lambda i: (i, 0, 0, 0)),
            pl.BlockSpec((1, 512), lambda i: (0, 0)),
            pl.BlockSpec((1, 512), lambda i: (0, 0)),
        ],
        out_specs=[
            pl.BlockSpec((8, 7, 7, 512), lambda i: (i, 0, 0, 0)),
            pl.BlockSpec((8, 512), lambda i: (i, 0)),
        ],
        out_shape=[
            jax.ShapeDtypeStruct((128, 7, 7, 512), F32),
            jax.ShapeDtypeStruct((128, 512), F32),
        ],
    )(y, scale, shift)


def _bn_inline(x2d, g, b):
    m = x2d.mean(axis=0)
    v = ((x2d - m) ** 2).mean(axis=0)
    scale = g * jax.lax.rsqrt(v + EPS)
    return x2d * scale + (b - m * scale)


def _temp_body(x_ref, w1_ref, w2_ref, w3_ref,
               g1_ref, b1_ref, g2_ref, b2_ref, g3_ref, b3_ref,
               feat_ref, emb_ref):
    xp = x_ref[...]
    acc = jnp.zeros((4000, 512), F32)
    for j in range(3):
        xs = xp[:, 2 * j:2 * j + 125, :, :].reshape(4000, 512)
        acc = acc + jnp.dot(xs, w1_ref[j], preferred_element_type=F32)
    y1 = _bn_inline(acc, g1_ref[0], b1_ref[0]).reshape(16, 125, 2, 512)
    y1p = y1[:, 0:124].reshape(16, 62, 2, 2, 512).max(axis=2)
    x2 = jnp.maximum(y1p, 0.0)
    y2 = (jnp.dot(x2[:, :, 0, :].reshape(992, 512), w2_ref[0],
                  preferred_element_type=F32)
          + jnp.dot(x2[:, :, 1, :].reshape(992, 512), w2_ref[1],
                    preferred_element_type=F32))
    x3 = jnp.maximum(_bn_inline(y2, g2_ref[0], b2_ref[0]), 0.0)
    x3 = x3.reshape(16, 62, 512)
    zrow = jnp.zeros((16, 1, 512), F32)
    x3p = jnp.concatenate([zrow, x3, zrow], axis=1)
    acc3 = jnp.zeros((992, 512), F32)
    for j in range(3):
        xs = x3p[:, j:j + 62, :].reshape(992, 512)
        acc3 = acc3 + jnp.dot(xs, w3_ref[j], preferred_element_type=F32)
    y3 = _bn_inline(acc3, g3_ref[0], b3_ref[0]).reshape(16, 62, 512)
    h = jnp.maximum(y3.reshape(16, 31, 2, 512).max(axis=2), 0.0)
    feat_ref[...] = h
    emb_ref[...] = h.max(axis=1)


def _temp_call(x_pad, w1, w2, w3, g1, b1, g2, b2, g3, b3):
    full = lambda shape: pl.BlockSpec(shape, lambda: tuple(0 for _ in shape))
    return pl.pallas_call(
        _temp_body,
        grid=(),
        in_specs=[
            full((16, 129, 2, 512)),
            full((3, 512, 512)), full((2, 512, 512)), full((3, 512, 512)),
            full((1, 512)), full((1, 512)), full((1, 512)),
            full((1, 512)), full((1, 512)), full((1, 512)),
        ],
        out_specs=[full((16, 31, 512)), full((16, 512))],
        out_shape=[
            jax.ShapeDtypeStruct((16, 31, 512), F32),
            jax.ShapeDtypeStruct((16, 512), F32),
        ],
    )(x_pad, w1, w2, w3, g1, b1, g2, b2, g3, b3)


def _mix_permutation():
    rng = np.random.default_rng(0)
    B, Fr = SEGMENTS, FRAMES
    randidx = np.zeros((B, Fr), dtype=np.int64)
    perm = np.zeros((B, Fr), dtype=np.int64)
    for seg in range(B):
        ri = rng.integers(0, B - 1, size=Fr)
        ri[ri >= seg] += 1
        randidx[seg] = ri
        perm[seg] = rng.permutation(Fr)
    flat = (randidx * Fr + perm).reshape(-1)
    P = np.zeros((B * Fr, B * Fr), dtype=np.float32)
    P[np.arange(B * Fr), flat] = 1.0
    return P


_MIX_P = _mix_permutation()


def _discrim_body(ea_ref, ev_ref, w1a_ref, w1b_ref, b1_ref, w2_ref, b2_ref,
                  p_ref, com_ref, dif_ref):
    A = jnp.dot(ea_ref[...], w1a_ref[...], preferred_element_type=F32)
    A_rep = jnp.broadcast_to(A[:, None, :], (16, 8, 128)).reshape(128, 128)
    Va = jnp.dot(ev_ref[...], w1b_ref[...], preferred_element_type=F32)
    b1 = b1_ref[0]
    hc = jnp.maximum(A_rep + Va + b1, 0.0)
    com_ref[...] = jnp.dot(hc, w2_ref[...], preferred_element_type=F32) + b2_ref[0]
    Vm = jnp.dot(p_ref[...], Va, preferred_element_type=F32)
    hd = jnp.maximum(A_rep + Vm + b1, 0.0)
    dif_ref[...] = jnp.dot(hd, w2_ref[...], preferred_element_type=F32) + b2_ref[0]


def _discrim_call(embed_a, embed_v, w1a, w1b, b1, w2, b2, P):
    full = lambda shape: pl.BlockSpec(shape, lambda: tuple(0 for _ in shape))
    return pl.pallas_call(
        _discrim_body,
        grid=(),
        in_specs=[
            full((16, 512)), full((128, 512)),
            full((512, 128)), full((512, 128)), full((1, 128)),
            full((128, 2)), full((1, 2)), full((128, 128)),
        ],
        out_specs=[full((128, 2)), full((128, 2))],
        out_shape=[
            jax.ShapeDtypeStruct((128, 2), F32),
            jax.ShapeDtypeStruct((128, 2), F32),
        ],
    )(embed_a, embed_v, w1a, w1b, b1, w2, b2, P)


def kernel(audio, visual, params):
    B, Fr = SEGMENTS, FRAMES

    caw2 = jnp.repeat(params['cls_a_w'], 2, axis=0) * 0.5
    a4 = audio[:, :500, :].reshape(B, 125, 4, 64).mean(axis=2)
    feat_a_t, pred_a3 = _mel_call(
        a4, params['mel_w'], params['mel_b'].reshape(1, -1),
        caw2, params['cls_a_b'].reshape(1, -1))
    pred_a = pred_a3.reshape(B, 15)

    v = visual.reshape(-1, 3, 7, 32, 7, 32)
    patches = jnp.transpose(v, (0, 2, 4, 1, 3, 5)).reshape(-1, 3072)
    feat_v, cam_flat, pred_v3 = _patch_call(
        patches, params['patch_w'], params['patch_b'].reshape(1, -1),
        params['cls_v_w'], params['cls_v_b'].reshape(1, -1))
    pred_v = pred_v3.reshape(B * Fr, 15)
    cam_v = jnp.transpose(cam_flat.reshape(B * Fr, 49, 15),
                          (0, 2, 1)).reshape(B * Fr, 15, 7, 7)

    fv4 = feat_v.reshape(B * Fr, 7, 7, 512)
    fv_pad = jnp.pad(fv4, ((0, 0), (1, 1), (1, 1), (0, 0)))
    w9 = jnp.transpose(params['sconv'], (2, 3, 1, 0)).reshape(9, 512, 512)
    y_raw, ssum, ssq = _spaconv_call(fv_pad, w9)
    n_sp = float(B * Fr * 49)
    mean = ssum.sum(axis=(0, 1)) / n_sp
    var = ssq.sum(axis=(0, 1)) / n_sp - mean * mean
    scale = params['sbn_g'] * jax.lax.rsqrt(var + EPS)
    shift = params['sbn_b'] - mean * scale
    fvh4, embed_v = _spabn_call(y_raw, scale.reshape(1, -1),
                                shift.reshape(1, -1))
    feat_v_h = jnp.transpose(fvh4, (0, 3, 1, 2))

    xa = jnp.transpose(feat_a_t.reshape(B, 125, 512, 2), (0, 1, 3, 2))
    xa_pad = jnp.pad(xa, ((0, 0), (2, 2), (0, 0), (0, 0)))
    w1 = jnp.transpose(params['tconv1'], (2, 3, 1, 0)).reshape(3, 512, 512)
    w2 = jnp.transpose(params['tconv2'], (2, 3, 1, 0)).reshape(2, 512, 512)
    w3 = jnp.transpose(params['tconv3'], (2, 3, 1, 0)).reshape(3, 512, 512)
    r = lambda p: params[p].reshape(1, -1)
    feat_a_hb, embed_a = _temp_call(
        xa_pad, w1, w2, w3,
        r('tbn1_g'), r('tbn1_b'), r('tbn2_g'), r('tbn2_b'),
        r('tbn3_g'), r('tbn3_b'))
    feat_a_h = jnp.transpose(feat_a_hb, (0, 2, 1))[:, :, :, None]

    common_f, differ_f = _discrim_call(
        embed_a, embed_v,
        params['d_w1'][:512], params['d_w1'][512:],
        params['d_b1'].reshape(1, -1), params['d_w2'],
        params['d_b2'].reshape(1, -1), jnp.asarray(_MIX_P))
    common = common_f.reshape(B, Fr, 2)
    differ = differ_f.reshape(B, Fr, 2)

    return common, differ, feat_a_h, feat_v_h, pred_a, pred_v, cam_v


# speedup vs baseline: 1.1357x; 1.1357x over previous
import numpy as np
import jax
import jax.numpy as jnp
from jax.experimental import pallas as pl

F32 = jnp.float32
FRAMES = 8
SEGMENTS = 16
EPS = 1e-5


def _mel_body(x_ref, w_ref, b_ref, caw_ref, cab_ref, feat_ref, pred_ref):
    x = x_ref[0]
    y4 = jnp.dot(x, w_ref[...], preferred_element_type=F32) + b_ref[0]
    feat_ref[0] = y4
    pooled = y4.mean(axis=0).reshape(1, 1024)
    pred_ref[0, 0] = (
        jnp.dot(pooled, caw_ref[...], preferred_element_type=F32)[0]
        + cab_ref[0]
    )


def _mel_call(a4, mel_w, mel_b, caw, cab):
    return pl.pallas_call(
        _mel_body,
        grid=(SEGMENTS,),
        in_specs=[
            pl.BlockSpec((1, 125, 64), lambda i: (i, 0, 0)),
            pl.BlockSpec((64, 1024), lambda i: (0, 0)),
            pl.BlockSpec((1, 1024), lambda i: (0, 0)),
            pl.BlockSpec((1024, 15), lambda i: (0, 0)),
            pl.BlockSpec((1, 15), lambda i: (0, 0)),
        ],
        out_specs=[
            pl.BlockSpec((1, 125, 1024), lambda i: (i, 0, 0)),
            pl.BlockSpec((1, 1, 15), lambda i: (i, 0, 0)),
        ],
        out_shape=[
            jax.ShapeDtypeStruct((SEGMENTS, 125, 1024), F32),
            jax.ShapeDtypeStruct((SEGMENTS, 1, 15), F32),
        ],
    )(a4, mel_w, mel_b, caw, cab)


def _vis_body(x_ref, w_ref, b_ref, cw_ref, cb_ref, w9_ref,
              y_ref, cam_ref, pv_ref, sum_ref, sq_ref):
    x = x_ref[...]
    v = jnp.dot(x, w_ref[...], preferred_element_type=F32) + b_ref[0]
    v = jnp.maximum(v, 0.0)
    cam = jnp.dot(v, cw_ref[...], preferred_element_type=F32) + cb_ref[0]
    pv_ref[0] = cam.reshape(8, 49, 15).mean(axis=1)
    cam_ref[...] = jnp.transpose(
        jnp.maximum(cam, 0.0).reshape(8, 49, 15), (0, 2, 1))
    fv4 = v.reshape(8, 7, 7, 512)
    zc = jnp.zeros((8, 7, 1, 512), F32)
    fvw = jnp.concatenate([zc, fv4, zc], axis=2)
    zr = jnp.zeros((8, 1, 9, 512), F32)
    fvp = jnp.concatenate([zr, fvw, zr], axis=1)
    acc = jnp.zeros((392, 512), F32)
    for k in range(9):
        dh, dw = k // 3, k % 3
        xs = fvp[:, dh:dh + 7, dw:dw + 7, :].reshape(392, 512)
        acc = acc + jnp.dot(xs, w9_ref[k], preferred_element_type=F32)
    y_ref[...] = acc.reshape(8, 49, 512)
    sum_ref[0, 0] = acc.sum(axis=0)
    sq_ref[0, 0] = (acc * acc).sum(axis=0)


def _vis_call(patches, patch_w, patch_b, cls_v_w, cls_v_b, w9):
    return pl.pallas_call(
        _vis_body,
        grid=(16,),
        in_specs=[
            pl.BlockSpec((392, 3072), lambda i: (i, 0)),
            pl.BlockSpec((3072, 512), lambda i: (0, 0)),
            pl.BlockSpec((1, 512), lambda i: (0, 0)),
            pl.BlockSpec((512, 15), lambda i: (0, 0)),
            pl.BlockSpec((1, 15), lambda i: (0, 0)),
            pl.BlockSpec((9, 512, 512), lambda i: (0, 0, 0)),
        ],
        out_specs=[
            pl.BlockSpec((8, 49, 512), lambda i: (i, 0, 0)),
            pl.BlockSpec((8, 15, 49), lambda i: (i, 0, 0)),
            pl.BlockSpec((1, 8, 15), lambda i: (i, 0, 0)),
            pl.BlockSpec((1, 1, 512), lambda i: (i, 0, 0)),
            pl.BlockSpec((1, 1, 512), lambda i: (i, 0, 0)),
        ],
        out_shape=[
            jax.ShapeDtypeStruct((128, 49, 512), F32),
            jax.ShapeDtypeStruct((128, 15, 49), F32),
            jax.ShapeDtypeStruct((16, 8, 15), F32),
            jax.ShapeDtypeStruct((16, 1, 512), F32),
            jax.ShapeDtypeStruct((16, 1, 512), F32),
        ],
    )(patches, patch_w, patch_b, cls_v_w, cls_v_b, w9)


def _spabn_body(y_ref, sc_ref, sh_ref, out_ref, emb_ref):
    y = y_ref[...]
    h = jnp.maximum(y * sc_ref[0] + sh_ref[0], 0.0)
    emb_ref[...] = h.max(axis=1)
    out_ref[...] = jnp.transpose(h, (0, 2, 1))


def _spabn_call(y, scale, shift):
    return pl.pallas_call(
        _spabn_body,
        grid=(16,),
        in_specs=[
            pl.BlockSpec((8, 49, 512), lambda i: (i, 0, 0)),
            pl.BlockSpec((1, 512), lambda i: (0, 0)),
            pl.BlockSpec((1, 512), lambda i: (0, 0)),
        ],
        out_specs=[
            pl.BlockSpec((8, 512, 49), lambda i: (i, 0, 0)),
            pl.BlockSpec((8, 512), lambda i: (i, 0)),
        ],
        out_shape=[
            jax.ShapeDtypeStruct((128, 512, 49), F32),
            jax.ShapeDtypeStruct((128, 512), F32),
        ],
    )(y, scale, shift)


def _bn_inline(x2d, g, b):
    m = x2d.mean(axis=0)
    v = ((x2d - m) ** 2).mean(axis=0)
    scale = g * jax.lax.rsqrt(v + EPS)
    return x2d * scale + (b - m * scale)


def _temp_body(x_ref, w1_ref, w2_ref, w3_ref,
               g1_ref, b1_ref, g2_ref, b2_ref, g3_ref, b3_ref,
               feat_ref, emb_ref):
    x = x_ref[...]
    z2 = jnp.zeros((16, 2, 2, 512), F32)
    xp = jnp.concatenate([z2, x, z2], axis=1)
    acc = jnp.zeros((4000, 512), F32)
    for j in range(3):
        xs = xp[:, 2 * j:2 * j + 125, :, :].reshape(4000, 512)
        acc = acc + jnp.dot(xs, w1_ref[j], preferred_element_type=F32)
    y1 = _bn_inline(acc, g1_ref[0], b1_ref[0]).reshape(16, 125, 2, 512)
    y1p = y1[:, 0:124].reshape(16, 62, 2, 2, 512).max(axis=2)
    x2 = jnp.maximum(y1p, 0.0)
    y2 = (jnp.dot(x2[:, :, 0, :].reshape(992, 512), w2_ref[0],
                  preferred_element_type=F32)
          + jnp.dot(x2[:, :, 1, :].reshape(992, 512), w2_ref[1],
                    preferred_element_type=F32))
    x3 = jnp.maximum(_bn_inline(y2, g2_ref[0], b2_ref[0]), 0.0)
    x3 = x3.reshape(16, 62, 512)
    zrow = jnp.zeros((16, 1, 512), F32)
    x3p = jnp.concatenate([zrow, x3, zrow], axis=1)
    acc3 = jnp.zeros((992, 512), F32)
    for j in range(3):
        xs = x3p[:, j:j + 62, :].reshape(992, 512)
        acc3 = acc3 + jnp.dot(xs, w3_ref[j], preferred_element_type=F32)
    y3 = _bn_inline(acc3, g3_ref[0], b3_ref[0]).reshape(16, 62, 512)
    h = jnp.maximum(y3.reshape(16, 31, 2, 512).max(axis=2), 0.0)
    feat_ref[...] = h
    emb_ref[...] = h.max(axis=1)


def _temp_call(x, w1, w2, w3, g1, b1, g2, b2, g3, b3):
    full = lambda shape: pl.BlockSpec(shape, lambda: tuple(0 for _ in shape))
    return pl.pallas_call(
        _temp_body,
        grid=(),
        in_specs=[
            full((16, 125, 2, 512)),
            full((3, 512, 512)), full((2, 512, 512)), full((3, 512, 512)),
            full((1, 512)), full((1, 512)), full((1, 512)),
            full((1, 512)), full((1, 512)), full((1, 512)),
        ],
        out_specs=[full((16, 31, 512)), full((16, 512))],
        out_shape=[
            jax.ShapeDtypeStruct((16, 31, 512), F32),
            jax.ShapeDtypeStruct((16, 512), F32),
        ],
    )(x, w1, w2, w3, g1, b1, g2, b2, g3, b3)


def _mix_permutation():
    rng = np.random.default_rng(0)
    B, Fr = SEGMENTS, FRAMES
    randidx = np.zeros((B, Fr), dtype=np.int64)
    perm = np.zeros((B, Fr), dtype=np.int64)
    for seg in range(B):
        ri = rng.integers(0, B - 1, size=Fr)
        ri[ri >= seg] += 1
        randidx[seg] = ri
        perm[seg] = rng.permutation(Fr)
    flat = (randidx * Fr + perm).reshape(-1)
    P = np.zeros((B * Fr, B * Fr), dtype=np.float32)
    P[np.arange(B * Fr), flat] = 1.0
    return P


_MIX_P = _mix_permutation()


def _discrim_body(ea_ref, ev_ref, w1a_ref, w1b_ref, b1_ref, w2_ref, b2_ref,
                  p_ref, com_ref, dif_ref):
    A = jnp.dot(ea_ref[...], w1a_ref[...], preferred_element_type=F32)
    A_rep = jnp.broadcast_to(A[:, None, :], (16, 8, 128)).reshape(128, 128)
    Va = jnp.dot(ev_ref[...], w1b_ref[...], preferred_element_type=F32)
    b1 = b1_ref[0]
    hc = jnp.maximum(A_rep + Va + b1, 0.0)
    com_ref[...] = jnp.dot(hc, w2_ref[...], preferred_element_type=F32) + b2_ref[0]
    Vm = jnp.dot(p_ref[...], Va, preferred_element_type=F32)
    hd = jnp.maximum(A_rep + Vm + b1, 0.0)
    dif_ref[...] = jnp.dot(hd, w2_ref[...], preferred_element_type=F32) + b2_ref[0]


def _discrim_call(embed_a, embed_v, w1a, w1b, b1, w2, b2, P):
    full = lambda shape: pl.BlockSpec(shape, lambda: tuple(0 for _ in shape))
    return pl.pallas_call(
        _discrim_body,
        grid=(),
        in_specs=[
            full((16, 512)), full((128, 512)),
            full((512, 128)), full((512, 128)), full((1, 128)),
            full((128, 2)), full((1, 2)), full((128, 128)),
        ],
        out_specs=[full((128, 2)), full((128, 2))],
        out_shape=[
            jax.ShapeDtypeStruct((128, 2), F32),
            jax.ShapeDtypeStruct((128, 2), F32),
        ],
    )(embed_a, embed_v, w1a, w1b, b1, w2, b2, P)


def kernel(audio, visual, params):
    B, Fr = SEGMENTS, FRAMES

    a4 = audio[:, :500, :].reshape(B, 125, 4, 64).mean(axis=2)
    mel_wp = params['mel_w'].reshape(64, 512, 2).transpose(0, 2, 1).reshape(64, 1024)
    mel_bp = params['mel_b'].reshape(512, 2).transpose(1, 0).reshape(1, 1024)
    caw2 = jnp.concatenate([params['cls_a_w'], params['cls_a_w']], axis=0) * 0.5
    feat_a_t, pred_a3 = _mel_call(
        a4, mel_wp, mel_bp, caw2, params['cls_a_b'].reshape(1, -1))
    pred_a = pred_a3.reshape(B, 15)

    v6 = visual.reshape(-1, 3, 7, 32, 7, 32)
    patches = jnp.transpose(v6, (0, 2, 4, 1, 3, 5)).reshape(-1, 3072)
    w9 = jnp.transpose(params['sconv'], (2, 3, 1, 0)).reshape(9, 512, 512)
    y_raw, cam49, pred_v3, ssum, ssq = _vis_call(
        patches, params['patch_w'], params['patch_b'].reshape(1, -1),
        params['cls_v_w'], params['cls_v_b'].reshape(1, -1), w9)
    pred_v = pred_v3.reshape(B * Fr, 15)
    cam_v = cam49.reshape(B * Fr, 15, 7, 7)

    n_sp = float(B * Fr * 49)
    mean = ssum.sum(axis=(0, 1)) / n_sp
    var = ssq.sum(axis=(0, 1)) / n_sp - mean * mean
    scale = params['sbn_g'] * jax.lax.rsqrt(var + EPS)
    shift = params['sbn_b'] - mean * scale
    fvh49, embed_v = _spabn_call(y_raw, scale.reshape(1, -1),
                                 shift.reshape(1, -1))
    feat_v_h = fvh49.reshape(B * Fr, 512, 7, 7)

    xa = feat_a_t.reshape(B, 125, 2, 512)
    w1 = jnp.transpose(params['tconv1'], (2, 3, 1, 0)).reshape(3, 512, 512)
    w2 = jnp.transpose(params['tconv2'], (2, 3, 1, 0)).reshape(2, 512, 512)
    w3 = jnp.transpose(params['tconv3'], (2, 3, 1, 0)).reshape(3, 512, 512)
    r = lambda p: params[p].reshape(1, -1)
    feat_a_hb, embed_a = _temp_call(
        xa, w1, w2, w3,
        r('tbn1_g'), r('tbn1_b'), r('tbn2_g'), r('tbn2_b'),
        r('tbn3_g'), r('tbn3_b'))
    feat_a_h = jnp.transpose(feat_a_hb, (0, 2, 1))[:, :, :, None]

    common_f, differ_f = _discrim_call(
        embed_a, embed_v,
        params['d_w1'][:512], params['d_w1'][512:],
        params['d_b1'].reshape(1, -1), params['d_w2'],
        params['d_b2'].reshape(1, -1), jnp.asarray(_MIX_P))
    common = common_f.reshape(B, Fr, 2)
    differ = differ_f.reshape(B, Fr, 2)

    return common, differ, feat_a_h, feat_v_h, pred_a, pred_v, cam_v


# speedup vs baseline: 1.4737x; 1.2976x over previous
import numpy as np
import jax
import jax.numpy as jnp
from jax.experimental import pallas as pl

F32 = jnp.float32
FRAMES = 8
SEGMENTS = 16
EPS = 1e-5


def _mel_body(x_ref, w_ref, b_ref, caw_ref, cab_ref, feat_ref, pred_ref):
    x = x_ref[0]
    y4 = jnp.dot(x, w_ref[...], preferred_element_type=F32) + b_ref[0]
    feat_ref[0] = y4
    pooled = y4.mean(axis=0).reshape(1, 1024)
    pred_ref[0, 0] = (
        jnp.dot(pooled, caw_ref[...], preferred_element_type=F32)[0]
        + cab_ref[0]
    )


def _mel_call(a4, mel_w, mel_b, caw, cab):
    return pl.pallas_call(
        _mel_body,
        grid=(SEGMENTS,),
        in_specs=[
            pl.BlockSpec((1, 125, 64), lambda i: (i, 0, 0)),
            pl.BlockSpec((64, 1024), lambda i: (0, 0)),
            pl.BlockSpec((1, 1024), lambda i: (0, 0)),
            pl.BlockSpec((1024, 15), lambda i: (0, 0)),
            pl.BlockSpec((1, 15), lambda i: (0, 0)),
        ],
        out_specs=[
            pl.BlockSpec((1, 125, 1024), lambda i: (i, 0, 0)),
            pl.BlockSpec((1, 1, 15), lambda i: (i, 0, 0)),
        ],
        out_shape=[
            jax.ShapeDtypeStruct((SEGMENTS, 125, 1024), F32),
            jax.ShapeDtypeStruct((SEGMENTS, 1, 15), F32),
        ],
    )(a4, mel_w, mel_b, caw, cab)


def _vis_body(x_ref, w_ref, b_ref, cw_ref, cb_ref, w9_ref,
              y_ref, cam_ref, pv_ref, sum_ref, sq_ref):
    x = x_ref[...]
    v3 = jax.lax.dot_general(x, w_ref[...], (((2,), (0,)), ((), ())),
                             preferred_element_type=F32) + b_ref[0]
    v3 = jnp.maximum(v3, 0.0)
    cam3 = jax.lax.dot_general(v3, cw_ref[...], (((2,), (0,)), ((), ())),
                               preferred_element_type=F32) + cb_ref[0]
    pv_ref[0] = cam3.mean(axis=1)
    cam_ref[...] = jnp.transpose(jnp.maximum(cam3, 0.0), (0, 2, 1))
    fv4 = v3.reshape(8, 7, 7, 512)
    zc = jnp.zeros((8, 7, 1, 512), F32)
    fvw = jnp.concatenate([zc, fv4, zc], axis=2)
    zr = jnp.zeros((8, 1, 9, 512), F32)
    fvp = jnp.concatenate([zr, fvw, zr], axis=1)
    acc = jnp.zeros((392, 512), F32)
    for k in range(9):
        dh, dw = k // 3, k % 3
        xs = fvp[:, dh:dh + 7, dw:dw + 7, :].reshape(392, 512)
        acc = acc + jnp.dot(xs, w9_ref[k], preferred_element_type=F32)
    y_ref[...] = acc.reshape(8, 49, 512)
    sum_ref[0, 0] = acc.sum(axis=0)
    sq_ref[0, 0] = (acc * acc).sum(axis=0)


def _vis_call(patches, patch_w, patch_b, cls_v_w, cls_v_b, w9):
    return pl.pallas_call(
        _vis_body,
        grid=(16,),
        in_specs=[
            pl.BlockSpec((8, 49, 3072), lambda i: (i, 0, 0)),
            pl.BlockSpec((3072, 512), lambda i: (0, 0)),
            pl.BlockSpec((1, 512), lambda i: (0, 0)),
            pl.BlockSpec((512, 15), lambda i: (0, 0)),
            pl.BlockSpec((1, 15), lambda i: (0, 0)),
            pl.BlockSpec((9, 512, 512), lambda i: (0, 0, 0)),
        ],
        out_specs=[
            pl.BlockSpec((8, 49, 512), lambda i: (i, 0, 0)),
            pl.BlockSpec((8, 15, 49), lambda i: (i, 0, 0)),
            pl.BlockSpec((1, 8, 15), lambda i: (i, 0, 0)),
            pl.BlockSpec((1, 1, 512), lambda i: (i, 0, 0)),
            pl.BlockSpec((1, 1, 512), lambda i: (i, 0, 0)),
        ],
        out_shape=[
            jax.ShapeDtypeStruct((128, 49, 512), F32),
            jax.ShapeDtypeStruct((128, 15, 49), F32),
            jax.ShapeDtypeStruct((16, 8, 15), F32),
            jax.ShapeDtypeStruct((16, 1, 512), F32),
            jax.ShapeDtypeStruct((16, 1, 512), F32),
        ],
    )(patches, patch_w, patch_b, cls_v_w, cls_v_b, w9)


def _spabn_body(y_ref, sc_ref, sh_ref, out_ref, emb_ref):
    y = y_ref[...]
    h = jnp.maximum(y * sc_ref[0] + sh_ref[0], 0.0)
    emb_ref[...] = h.max(axis=1)
    out_ref[...] = jnp.transpose(h, (0, 2, 1))


def _spabn_call(y, scale, shift):
    return pl.pallas_call(
        _spabn_body,
        grid=(16,),
        in_specs=[
            pl.BlockSpec((8, 49, 512), lambda i: (i, 0, 0)),
            pl.BlockSpec((1, 512), lambda i: (0, 0)),
            pl.BlockSpec((1, 512), lambda i: (0, 0)),
        ],
        out_specs=[
            pl.BlockSpec((8, 512, 49), lambda i: (i, 0, 0)),
            pl.BlockSpec((8, 512), lambda i: (i, 0)),
        ],
        out_shape=[
            jax.ShapeDtypeStruct((128, 512, 49), F32),
            jax.ShapeDtypeStruct((128, 512), F32),
        ],
    )(y, scale, shift)


def _bn_inline(x2d, g, b):
    m = x2d.mean(axis=0)
    v = ((x2d - m) ** 2).mean(axis=0)
    scale = g * jax.lax.rsqrt(v + EPS)
    return x2d * scale + (b - m * scale)


def _temp_body(x_ref, w1_ref, w2_ref, w3_ref,
               g1_ref, b1_ref, g2_ref, b2_ref, g3_ref, b3_ref,
               feat_ref, emb_ref):
    x = x_ref[...]
    z2 = jnp.zeros((16, 2, 2, 512), F32)
    xp = jnp.concatenate([z2, x, z2], axis=1)
    acc = jnp.zeros((4000, 512), F32)
    for j in range(3):
        xs = xp[:, 2 * j:2 * j + 125, :, :].reshape(4000, 512)
        acc = acc + jnp.dot(xs, w1_ref[j], preferred_element_type=F32)
    y1 = _bn_inline(acc, g1_ref[0], b1_ref[0]).reshape(16, 125, 2, 512)
    y1p = y1[:, 0:124].reshape(16, 62, 2, 2, 512).max(axis=2)
    x2 = jnp.maximum(y1p, 0.0)
    y2 = (jnp.dot(x2[:, :, 0, :].reshape(992, 512), w2_ref[0],
                  preferred_element_type=F32)
          + jnp.dot(x2[:, :, 1, :].reshape(992, 512), w2_ref[1],
                    preferred_element_type=F32))
    x3 = jnp.maximum(_bn_inline(y2, g2_ref[0], b2_ref[0]), 0.0)
    x3 = x3.reshape(16, 62, 512)
    zrow = jnp.zeros((16, 1, 512), F32)
    x3p = jnp.concatenate([zrow, x3, zrow], axis=1)
    acc3 = jnp.zeros((992, 512), F32)
    for j in range(3):
        xs = x3p[:, j:j + 62, :].reshape(992, 512)
        acc3 = acc3 + jnp.dot(xs, w3_ref[j], preferred_element_type=F32)
    y3 = _bn_inline(acc3, g3_ref[0], b3_ref[0]).reshape(16, 62, 512)
    h = jnp.maximum(y3.reshape(16, 31, 2, 512).max(axis=2), 0.0)
    feat_ref[...] = h
    emb_ref[...] = h.max(axis=1)


def _temp_call(x, w1, w2, w3, g1, b1, g2, b2, g3, b3):
    full = lambda shape: pl.BlockSpec(shape, lambda: tuple(0 for _ in shape))
    return pl.pallas_call(
        _temp_body,
        grid=(),
        in_specs=[
            full((16, 125, 2, 512)),
            full((3, 512, 512)), full((2, 512, 512)), full((3, 512, 512)),
            full((1, 512)), full((1, 512)), full((1, 512)),
            full((1, 512)), full((1, 512)), full((1, 512)),
        ],
        out_specs=[full((16, 31, 512)), full((16, 512))],
        out_shape=[
            jax.ShapeDtypeStruct((16, 31, 512), F32),
            jax.ShapeDtypeStruct((16, 512), F32),
        ],
    )(x, w1, w2, w3, g1, b1, g2, b2, g3, b3)


def _mix_permutation():
    rng = np.random.default_rng(0)
    B, Fr = SEGMENTS, FRAMES
    randidx = np.zeros((B, Fr), dtype=np.int64)
    perm = np.zeros((B, Fr), dtype=np.int64)
    for seg in range(B):
        ri = rng.integers(0, B - 1, size=Fr)
        ri[ri >= seg] += 1
        randidx[seg] = ri
        perm[seg] = rng.permutation(Fr)
    flat = (randidx * Fr + perm).reshape(-1)
    P = np.zeros((B * Fr, B * Fr), dtype=np.float32)
    P[np.arange(B * Fr), flat] = 1.0
    return P


_MIX_P = _mix_permutation()


def _discrim_body(ea_ref, ev_ref, w1a_ref, w1b_ref, b1_ref, w2_ref, b2_ref,
                  p_ref, com_ref, dif_ref):
    A = jnp.dot(ea_ref[...], w1a_ref[...], preferred_element_type=F32)
    A_rep = jnp.broadcast_to(A[:, None, :], (16, 8, 128)).reshape(128, 128)
    Va = jnp.dot(ev_ref[...], w1b_ref[...], preferred_element_type=F32)
    b1 = b1_ref[0]
    hc = jnp.maximum(A_rep + Va + b1, 0.0)
    com_ref[...] = jnp.dot(hc, w2_ref[...], preferred_element_type=F32) + b2_ref[0]
    Vm = jnp.dot(p_ref[...], Va, preferred_element_type=F32)
    hd = jnp.maximum(A_rep + Vm + b1, 0.0)
    dif_ref[...] = jnp.dot(hd, w2_ref[...], preferred_element_type=F32) + b2_ref[0]


def _discrim_call(embed_a, embed_v, w1a, w1b, b1, w2, b2, P):
    full = lambda shape: pl.BlockSpec(shape, lambda: tuple(0 for _ in shape))
    return pl.pallas_call(
        _discrim_body,
        grid=(),
        in_specs=[
            full((16, 512)), full((128, 512)),
            full((512, 128)), full((512, 128)), full((1, 128)),
            full((128, 2)), full((1, 2)), full((128, 128)),
        ],
        out_specs=[full((128, 2)), full((128, 2))],
        out_shape=[
            jax.ShapeDtypeStruct((128, 2), F32),
            jax.ShapeDtypeStruct((128, 2), F32),
        ],
    )(embed_a, embed_v, w1a, w1b, b1, w2, b2, P)


def kernel(audio, visual, params):
    B, Fr = SEGMENTS, FRAMES

    a4 = audio[:, :500, :].reshape(B, 125, 4, 64).mean(axis=2)
    mel_wp = params['mel_w'].reshape(64, 512, 2).transpose(0, 2, 1).reshape(64, 1024)
    mel_bp = params['mel_b'].reshape(512, 2).transpose(1, 0).reshape(1, 1024)
    caw2 = jnp.concatenate([params['cls_a_w'], params['cls_a_w']], axis=0) * 0.5
    feat_a_t, pred_a3 = _mel_call(
        a4, mel_wp, mel_bp, caw2, params['cls_a_b'].reshape(1, -1))
    pred_a = pred_a3.reshape(B, 15)

    v6 = visual.reshape(-1, 3, 7, 32, 7, 32)
    patches = jnp.transpose(v6, (0, 2, 4, 1, 3, 5)).reshape(-1, 49, 3072)
    w9 = jnp.transpose(params['sconv'], (2, 3, 1, 0)).reshape(9, 512, 512)
    y_raw, cam49, pred_v3, ssum, ssq = _vis_call(
        patches, params['patch_w'], params['patch_b'].reshape(1, -1),
        params['cls_v_w'], params['cls_v_b'].reshape(1, -1), w9)
    pred_v = pred_v3.reshape(B * Fr, 15)
    cam_v = cam49.reshape(B * Fr, 15, 7, 7)

    n_sp = float(B * Fr * 49)
    mean = ssum.sum(axis=(0, 1)) / n_sp
    var = ssq.sum(axis=(0, 1)) / n_sp - mean * mean
    scale = params['sbn_g'] * jax.lax.rsqrt(var + EPS)
    shift = params['sbn_b'] - mean * scale
    fvh49, embed_v = _spabn_call(y_raw, scale.reshape(1, -1),
                                 shift.reshape(1, -1))
    feat_v_h = fvh49.reshape(B * Fr, 512, 7, 7)

    xa = feat_a_t.reshape(B, 125, 2, 512)
    w1 = jnp.transpose(params['tconv1'], (2, 3, 1, 0)).reshape(3, 512, 512)
    w2 = jnp.transpose(params['tconv2'], (2, 3, 1, 0)).reshape(2, 512, 512)
    w3 = jnp.transpose(params['tconv3'], (2, 3, 1, 0)).reshape(3, 512, 512)
    r = lambda p: params[p].reshape(1, -1)
    feat_a_hb, embed_a = _temp_call(
        xa, w1, w2, w3,
        r('tbn1_g'), r('tbn1_b'), r('tbn2_g'), r('tbn2_b'),
        r('tbn3_g'), r('tbn3_b'))
    feat_a_h = jnp.transpose(feat_a_hb, (0, 2, 1))[:, :, :, None]

    common_f, differ_f = _discrim_call(
        embed_a, embed_v,
        params['d_w1'][:512], params['d_w1'][512:],
        params['d_b1'].reshape(1, -1), params['d_w2'],
        params['d_b2'].reshape(1, -1), jnp.asarray(_MIX_P))
    common = common_f.reshape(B, Fr, 2)
    differ = differ_f.reshape(B, Fr, 2)

    return common, differ, feat_a_h, feat_v_h, pred_a, pred_v, cam_v


# speedup vs baseline: 1.6154x; 1.0961x over previous
import numpy as np
import jax
import jax.numpy as jnp
from jax.experimental import pallas as pl

F32 = jnp.float32
FRAMES = 8
SEGMENTS = 16
EPS = 1e-5


def _mel_body(x_ref, w_ref, b_ref, caw_ref, cab_ref,
              x0_ref, x1_ref, pred_ref):
    x = x_ref[...]
    y = jnp.dot(x, w_ref[...], preferred_element_type=F32) + b_ref[0]
    x0_ref[...] = y[:, :512].reshape(16, 125, 512)
    x1_ref[...] = y[:, 512:].reshape(16, 125, 512)
    pooled = y.reshape(16, 125, 1024).mean(axis=1)
    pred_ref[...] = (
        jnp.dot(pooled, caw_ref[...], preferred_element_type=F32)
        + cab_ref[0]
    )


def _mel_call(a4, mel_w, mel_b, caw, cab):
    full = lambda shape: pl.BlockSpec(shape, lambda: tuple(0 for _ in shape))
    return pl.pallas_call(
        _mel_body,
        grid=(),
        in_specs=[
            full((2000, 64)), full((64, 1024)), full((1, 1024)),
            full((1024, 15)), full((1, 15)),
        ],
        out_specs=[
            full((16, 125, 512)), full((16, 125, 512)), full((16, 15)),
        ],
        out_shape=[
            jax.ShapeDtypeStruct((16, 125, 512), F32),
            jax.ShapeDtypeStruct((16, 125, 512), F32),
            jax.ShapeDtypeStruct((16, 15), F32),
        ],
    )(a4, mel_w, mel_b, caw, cab)


def _vis_body(x_ref, w_ref, b_ref, cw_ref, cb_ref, w9_ref,
              y_ref, cam_ref, pv_ref, sum_ref, sq_ref):
    x = x_ref[...]
    v3 = jax.lax.dot_general(x, w_ref[...], (((2,), (0,)), ((), ())),
                             preferred_element_type=F32) + b_ref[0]
    v3 = jnp.maximum(v3, 0.0)
    cam3 = jax.lax.dot_general(v3, cw_ref[...], (((2,), (0,)), ((), ())),
                               preferred_element_type=F32) + cb_ref[0]
    pv_ref[0] = cam3.mean(axis=1)
    cam_ref[...] = jnp.transpose(jnp.maximum(cam3, 0.0), (0, 2, 1))
    fv4 = v3.reshape(8, 7, 7, 512)
    zc = jnp.zeros((8, 7, 1, 512), F32)
    fvw = jnp.concatenate([zc, fv4, zc], axis=2)
    zr = jnp.zeros((8, 1, 9, 512), F32)
    fvp = jnp.concatenate([zr, fvw, zr], axis=1)
    acc = jnp.zeros((392, 512), F32)
    for k in range(9):
        dh, dw = k // 3, k % 3
        xs = fvp[:, dh:dh + 7, dw:dw + 7, :].reshape(392, 512)
        acc = acc + jnp.dot(xs, w9_ref[k], preferred_element_type=F32)
    y_ref[...] = acc.reshape(8, 49, 512)
    sum_ref[0, 0] = acc.sum(axis=0)
    sq_ref[0, 0] = (acc * acc).sum(axis=0)


def _vis_call(patches, patch_w, patch_b, cls_v_w, cls_v_b, w9):
    return pl.pallas_call(
        _vis_body,
        grid=(16,),
        in_specs=[
            pl.BlockSpec((8, 49, 3072), lambda i: (i, 0, 0)),
            pl.BlockSpec((3072, 512), lambda i: (0, 0)),
            pl.BlockSpec((1, 512), lambda i: (0, 0)),
            pl.BlockSpec((512, 15), lambda i: (0, 0)),
            pl.BlockSpec((1, 15), lambda i: (0, 0)),
            pl.BlockSpec((9, 512, 512), lambda i: (0, 0, 0)),
        ],
        out_specs=[
            pl.BlockSpec((8, 49, 512), lambda i: (i, 0, 0)),
            pl.BlockSpec((8, 15, 49), lambda i: (i, 0, 0)),
            pl.BlockSpec((1, 8, 15), lambda i: (i, 0, 0)),
            pl.BlockSpec((1, 1, 512), lambda i: (i, 0, 0)),
            pl.BlockSpec((1, 1, 512), lambda i: (i, 0, 0)),
        ],
        out_shape=[
            jax.ShapeDtypeStruct((128, 49, 512), F32),
            jax.ShapeDtypeStruct((128, 15, 49), F32),
            jax.ShapeDtypeStruct((16, 8, 15), F32),
            jax.ShapeDtypeStruct((16, 1, 512), F32),
            jax.ShapeDtypeStruct((16, 1, 512), F32),
        ],
    )(patches, patch_w, patch_b, cls_v_w, cls_v_b, w9)


def _spabn_body(y_ref, sc_ref, sh_ref, out_ref, emb_ref):
    y = y_ref[...]
    h = jnp.maximum(y * sc_ref[0] + sh_ref[0], 0.0)
    emb_ref[...] = h.max(axis=1)
    out_ref[...] = h


def _spabn_call(y, scale, shift):
    return pl.pallas_call(
        _spabn_body,
        grid=(16,),
        in_specs=[
            pl.BlockSpec((8, 49, 512), lambda i: (i, 0, 0)),
            pl.BlockSpec((1, 512), lambda i: (0, 0)),
            pl.BlockSpec((1, 512), lambda i: (0, 0)),
        ],
        out_specs=[
            pl.BlockSpec((8, 49, 512), lambda i: (i, 0, 0)),
            pl.BlockSpec((8, 512), lambda i: (i, 0)),
        ],
        out_shape=[
            jax.ShapeDtypeStruct((128, 49, 512), F32),
            jax.ShapeDtypeStruct((128, 512), F32),
        ],
    )(y, scale, shift)


def _bn_inline(x2d, g, b):
    m = x2d.mean(axis=0)
    v = ((x2d - m) ** 2).mean(axis=0)
    scale = g * jax.lax.rsqrt(v + EPS)
    return x2d * scale + (b - m * scale)


def _temp_body(x0_ref, x1_ref, w1_ref, w2_ref, w3_ref,
               g1_ref, b1_ref, g2_ref, b2_ref, g3_ref, b3_ref,
               feat_ref, emb_ref):
    z2 = jnp.zeros((16, 2, 512), F32)
    accs = []
    for x_ref in (x0_ref, x1_ref):
        xp = jnp.concatenate([z2, x_ref[...], z2], axis=1)
        acc = jnp.zeros((2000, 512), F32)
        for j in range(3):
            xs = xp[:, 2 * j:2 * j + 125, :].reshape(2000, 512)
            acc = acc + jnp.dot(xs, w1_ref[j], preferred_element_type=F32)
        accs.append(acc)
    m = (accs[0].sum(axis=0) + accs[1].sum(axis=0)) * (1.0 / 4000.0)
    v = (((accs[0] - m) ** 2).sum(axis=0)
         + ((accs[1] - m) ** 2).sum(axis=0)) * (1.0 / 4000.0)
    scale1 = g1_ref[0] * jax.lax.rsqrt(v + EPS)
    shift1 = b1_ref[0] - m * scale1
    x2 = []
    for acc in accs:
        y1 = (acc * scale1 + shift1).reshape(16, 125, 512)
        y1p = y1[:, 0:124].reshape(16, 62, 2, 512).max(axis=2)
        x2.append(jnp.maximum(y1p, 0.0).reshape(992, 512))
    y2 = (jnp.dot(x2[0], w2_ref[0], preferred_element_type=F32)
          + jnp.dot(x2[1], w2_ref[1], preferred_element_type=F32))
    x3 = jnp.maximum(_bn_inline(y2, g2_ref[0], b2_ref[0]), 0.0)
    x3 = x3.reshape(16, 62, 512)
    zrow = jnp.zeros((16, 1, 512), F32)
    x3p = jnp.concatenate([zrow, x3, zrow], axis=1)
    acc3 = jnp.zeros((992, 512), F32)
    for j in range(3):
        xs = x3p[:, j:j + 62, :].reshape(992, 512)
        acc3 = acc3 + jnp.dot(xs, w3_ref[j], preferred_element_type=F32)
    y3 = _bn_inline(acc3, g3_ref[0], b3_ref[0]).reshape(16, 62, 512)
    h = jnp.maximum(y3.reshape(16, 31, 2, 512).max(axis=2), 0.0)
    feat_ref[...] = h
    emb_ref[...] = h.max(axis=1)


def _temp_call(x0, x1, w1, w2, w3, g1, b1, g2, b2, g3, b3):
    full = lambda shape: pl.BlockSpec(shape, lambda: tuple(0 for _ in shape))
    return pl.pallas_call(
        _temp_body,
        grid=(),
        in_specs=[
            full((16, 125, 512)), full((16, 125, 512)),
            full((3, 512, 512)), full((2, 512, 512)), full((3, 512, 512)),
            full((1, 512)), full((1, 512)), full((1, 512)),
            full((1, 512)), full((1, 512)), full((1, 512)),
        ],
        out_specs=[full((16, 31, 512)), full((16, 512))],
        out_shape=[
            jax.ShapeDtypeStruct((16, 31, 512), F32),
            jax.ShapeDtypeStruct((16, 512), F32),
        ],
    )(x0, x1, w1, w2, w3, g1, b1, g2, b2, g3, b3)


def _mix_permutation():
    rng = np.random.default_rng(0)
    B, Fr = SEGMENTS, FRAMES
    randidx = np.zeros((B, Fr), dtype=np.int64)
    perm = np.zeros((B, Fr), dtype=np.int64)
    for seg in range(B):
        ri = rng.integers(0, B - 1, size=Fr)
        ri[ri >= seg] += 1
        randidx[seg] = ri
        perm[seg] = rng.permutation(Fr)
    flat = (randidx * Fr + perm).reshape(-1)
    P = np.zeros((B * Fr, B * Fr), dtype=np.float32)
    P[np.arange(B * Fr), flat] = 1.0
    return P


_MIX_P = _mix_permutation()


def _discrim_body(ea_ref, ev_ref, w1a_ref, w1b_ref, b1_ref, w2_ref, b2_ref,
                  p_ref, com_ref, dif_ref):
    A = jnp.dot(ea_ref[...], w1a_ref[...], preferred_element_type=F32)
    A_rep = jnp.broadcast_to(A[:, None, :], (16, 8, 128)).reshape(128, 128)
    Va = jnp.dot(ev_ref[...], w1b_ref[...], preferred_element_type=F32)
    b1 = b1_ref[0]
    hc = jnp.maximum(A_rep + Va + b1, 0.0)
    com_ref[...] = jnp.dot(hc, w2_ref[...], preferred_element_type=F32) + b2_ref[0]
    Vm = jnp.dot(p_ref[...], Va, preferred_element_type=F32)
    hd = jnp.maximum(A_rep + Vm + b1, 0.0)
    dif_ref[...] = jnp.dot(hd, w2_ref[...], preferred_element_type=F32) + b2_ref[0]


def _discrim_call(embed_a, embed_v, w1a, w1b, b1, w2, b2, P):
    full = lambda shape: pl.BlockSpec(shape, lambda: tuple(0 for _ in shape))
    return pl.pallas_call(
        _discrim_body,
        grid=(),
        in_specs=[
            full((16, 512)), full((128, 512)),
            full((512, 128)), full((512, 128)), full((1, 128)),
            full((128, 2)), full((1, 2)), full((128, 128)),
        ],
        out_specs=[full((128, 2)), full((128, 2))],
        out_shape=[
            jax.ShapeDtypeStruct((128, 2), F32),
            jax.ShapeDtypeStruct((128, 2), F32),
        ],
    )(embed_a, embed_v, w1a, w1b, b1, w2, b2, P)


def kernel(audio, visual, params):
    B, Fr = SEGMENTS, FRAMES

    a4 = audio[:, :500, :].reshape(B, 125, 4, 64).mean(axis=2).reshape(2000, 64)
    mel_wp = params['mel_w'].reshape(64, 512, 2).transpose(0, 2, 1).reshape(64, 1024)
    mel_bp = params['mel_b'].reshape(512, 2).transpose(1, 0).reshape(1, 1024)
    caw2 = jnp.concatenate([params['cls_a_w'], params['cls_a_w']], axis=0) * 0.5
    xa0, xa1, pred_a = _mel_call(
        a4, mel_wp, mel_bp, caw2, params['cls_a_b'].reshape(1, -1))

    v6 = visual.reshape(-1, 3, 7, 32, 7, 32)
    patches = jnp.transpose(v6, (0, 2, 4, 1, 3, 5)).reshape(-1, 49, 3072)
    w9 = jnp.transpose(params['sconv'], (2, 3, 1, 0)).reshape(9, 512, 512)
    y_raw, cam49, pred_v3, ssum, ssq = _vis_call(
        patches, params['patch_w'], params['patch_b'].reshape(1, -1),
        params['cls_v_w'], params['cls_v_b'].reshape(1, -1), w9)
    pred_v = pred_v3.reshape(B * Fr, 15)
    cam_v = cam49.reshape(B * Fr, 15, 7, 7)

    n_sp = float(B * Fr * 49)
    mean = ssum.sum(axis=(0, 1)) / n_sp
    var = ssq.sum(axis=(0, 1)) / n_sp - mean * mean
    scale = params['sbn_g'] * jax.lax.rsqrt(var + EPS)
    shift = params['sbn_b'] - mean * scale
    fvh49, embed_v = _spabn_call(y_raw, scale.reshape(1, -1),
                                 shift.reshape(1, -1))
    feat_v_h = jnp.transpose(fvh49.reshape(B * Fr, 7, 7, 512), (0, 3, 1, 2))

    w1 = jnp.transpose(params['tconv1'], (2, 3, 1, 0)).reshape(3, 512, 512)
    w2 = jnp.transpose(params['tconv2'], (2, 3, 1, 0)).reshape(2, 512, 512)
    w3 = jnp.transpose(params['tconv3'], (2, 3, 1, 0)).reshape(3, 512, 512)
    r = lambda p: params[p].reshape(1, -1)
    feat_a_hb, embed_a = _temp_call(
        xa0, xa1, w1, w2, w3,
        r('tbn1_g'), r('tbn1_b'), r('tbn2_g'), r('tbn2_b'),
        r('tbn3_g'), r('tbn3_b'))
    feat_a_h = jnp.transpose(feat_a_hb, (0, 2, 1))[:, :, :, None]

    common_f, differ_f = _discrim_call(
        embed_a, embed_v,
        params['d_w1'][:512], params['d_w1'][512:],
        params['d_b1'].reshape(1, -1), params['d_w2'],
        params['d_b2'].reshape(1, -1), jnp.asarray(_MIX_P))
    common = common_f.reshape(B, Fr, 2)
    differ = differ_f.reshape(B, Fr, 2)

    return common, differ, feat_a_h, feat_v_h, pred_a, pred_v, cam_v
